# baseline (speedup 1.0000x reference)
import functools

import jax
import jax.numpy as jnp
from jax import lax
from jax.experimental import pallas as pl
from jax.experimental.pallas import tpu as pltpu
from jax.experimental.pallas import tpu_sc as plsc


def _broadcast_sc(pos, batch_size):
    seq_len, embed_dim = pos.shape
    info = plsc.get_sparse_core_info()
    num_cores, num_subcores = info.num_cores, info.num_subcores
    num_workers = num_cores * num_subcores
    rows_per_worker = seq_len // num_workers
    chunk = min(rows_per_worker, 32)
    n_chunks = rows_per_worker // chunk
    NBUF = 3
    AHEAD = 1

    mesh = plsc.VectorSubcoreMesh(core_axis_name="c", subcore_axis_name="s")

    scratch = (
        [pltpu.VMEM((chunk, embed_dim), pos.dtype) for _ in range(NBUF)]
        + [pltpu.SemaphoreType.DMA for _ in range(2 * NBUF)]
    )

    @functools.partial(
        pl.kernel,
        mesh=mesh,
        out_type=jax.ShapeDtypeStruct((batch_size, seq_len, embed_dim), pos.dtype),
        scratch_types=scratch,
    )
    def bcast(w_hbm, out_hbm, *refs):
        bufs = refs[:NBUF]
        rsems = refs[NBUF : 2 * NBUF]
        wsems = refs[2 * NBUF : 3 * NBUF]
        wid = lax.axis_index("s") * num_cores + lax.axis_index("c")
        base = wid * rows_per_worker

        def read(c):
            s = c % NBUF
            return pltpu.async_copy(
                w_hbm.at[pl.ds(base + c * chunk, chunk)], bufs[s], rsems[s]
            )

        def write(c):
            s = c % NBUF
            return [
                pltpu.async_copy(
                    bufs[s], out_hbm.at[b, pl.ds(base + c * chunk, chunk)], wsems[s]
                )
                for b in range(batch_size)
            ]

        pending_writes = [None] * NBUF
        pending_reads = [None] * n_chunks
        for c in range(min(AHEAD + 1, n_chunks)):
            pending_reads[c] = read(c)
        for c in range(n_chunks):
            nxt = c + AHEAD + 1
            if nxt < n_chunks:
                s = nxt % NBUF
                if pending_writes[s] is not None:
                    for h in pending_writes[s]:
                        h.wait()
                    pending_writes[s] = None
                pending_reads[nxt] = read(nxt)
            pending_reads[c].wait()
            pending_writes[c % NBUF] = write(c)
        for s in range(NBUF):
            if pending_writes[s] is not None:
                for h in pending_writes[s]:
                    h.wait()

    return bcast(pos)


def kernel(tokens, positional_embedding_weights):
    batch_size, seq_len = tokens.shape
    pos = positional_embedding_weights[:seq_len]
    return _broadcast_sc(pos, batch_size)


# speedup vs baseline: 1.5227x; 1.5227x over previous
import jax
import jax.numpy as jnp
from jax.experimental import pallas as pl
from jax.experimental.pallas import tpu as pltpu


def kernel(tokens, positional_embedding_weights):
    batch_size, seq_len = tokens.shape
    pos = positional_embedding_weights[:seq_len]
    S, D = pos.shape
    CH = 2048
    n_chunks = S // CH
    NBUF = 4
    AHEAD = 1

    def body(in_hbm, out_hbm, *refs):
        bufs = refs[:NBUF]
        rsems = refs[NBUF : 2 * NBUF]
        wsems = refs[2 * NBUF : 3 * NBUF]

        def read(c):
            s = c % NBUF
            cp = pltpu.make_async_copy(in_hbm.at[pl.ds(c * CH, CH)], bufs[s], rsems[s])
            cp.start()
            return cp

        def write(c):
            s = c % NBUF
            cps = []
            for b in range(batch_size):
                cp = pltpu.make_async_copy(
                    bufs[s], out_hbm.at[b, pl.ds(c * CH, CH)], wsems[s]
                )
                cp.start()
                cps.append(cp)
            return cps

        pending_writes = [None] * NBUF
        pending_reads = [None] * n_chunks
        for c in range(min(AHEAD + 1, n_chunks)):
            pending_reads[c] = read(c)
        for c in range(n_chunks):
            nxt = c + AHEAD + 1
            if nxt < n_chunks:
                s = nxt % NBUF
                if pending_writes[s] is not None:
                    for h in pending_writes[s]:
                        h.wait()
                    pending_writes[s] = None
                pending_reads[nxt] = read(nxt)
            pending_reads[c].wait()
            pending_writes[c % NBUF] = write(c)
        for s in range(NBUF):
            if pending_writes[s] is not None:
                for h in pending_writes[s]:
                    h.wait()

    scratch = (
        [pltpu.VMEM((CH, D), pos.dtype) for _ in range(NBUF)]
        + [pltpu.SemaphoreType.DMA for _ in range(2 * NBUF)]
    )
    return pl.pallas_call(
        body,
        in_specs=[pl.BlockSpec(memory_space=pltpu.MemorySpace.HBM)],
        out_specs=pl.BlockSpec(memory_space=pltpu.MemorySpace.HBM),
        out_shape=jax.ShapeDtypeStruct((batch_size, S, D), pos.dtype),
        scratch_shapes=scratch,
    )(pos)


# speedup vs baseline: 1.5283x; 1.0037x over previous
import jax
import jax.numpy as jnp
from jax.experimental import pallas as pl
from jax.experimental.pallas import tpu as pltpu


def kernel(tokens, positional_embedding_weights):
    batch_size, seq_len = tokens.shape
    pos = positional_embedding_weights[:seq_len]
    S, D = pos.shape
    CH = 2048
    n_chunks = S // CH
    NBUF = 4
    AHEAD = 3

    def body(in_hbm, out_hbm, *refs):
        bufs = refs[:NBUF]
        rsems = refs[NBUF : 2 * NBUF]
        wsems = refs[2 * NBUF : 3 * NBUF]

        def read(c):
            s = c % NBUF
            cp = pltpu.make_async_copy(in_hbm.at[pl.ds(c * CH, CH)], bufs[s], rsems[s])
            cp.start()
            return cp

        def write(c):
            s = c % NBUF
            cps = []
            for b in range(batch_size):
                cp = pltpu.make_async_copy(
                    bufs[s], out_hbm.at[b, pl.ds(c * CH, CH)], wsems[s]
                )
                cp.start()
                cps.append(cp)
            return cps

        pending_writes = [None] * NBUF
        pending_reads = [None] * n_chunks
        for c in range(min(AHEAD + 1, n_chunks)):
            pending_reads[c] = read(c)
        for c in range(n_chunks):
            nxt = c + AHEAD + 1
            if nxt < n_chunks:
                s = nxt % NBUF
                if pending_writes[s] is not None:
                    for h in pending_writes[s]:
                        h.wait()
                    pending_writes[s] = None
                pending_reads[nxt] = read(nxt)
            pending_reads[c].wait()
            pending_writes[c % NBUF] = write(c)
        for s in range(NBUF):
            if pending_writes[s] is not None:
                for h in pending_writes[s]:
                    h.wait()

    scratch = (
        [pltpu.VMEM((CH, D), pos.dtype) for _ in range(NBUF)]
        + [pltpu.SemaphoreType.DMA for _ in range(2 * NBUF)]
    )
    return pl.pallas_call(
        body,
        in_specs=[pl.BlockSpec(memory_space=pltpu.MemorySpace.HBM)],
        out_specs=pl.BlockSpec(memory_space=pltpu.MemorySpace.HBM),
        out_shape=jax.ShapeDtypeStruct((batch_size, S, D), pos.dtype),
        scratch_shapes=scratch,
    )(pos)
